# baseline (speedup 1.0000x reference)
import functools

import jax
import jax.numpy as jnp
from jax import lax
from jax.experimental import pallas as pl
from jax.experimental.pallas import tpu as pltpu
from jax.experimental.pallas import tpu_sc as plsc

N = 10000
D = 128
D_HID = 256

NC = 2
NS = 16
NW = NC * NS

N_PAD = 10240
ROWS_PER_TILE = N_PAD // NS
E_PAD = 327680
EPW = E_PAD // NW
CHUNK = 80
NCHUNK = EPW // CHUNK
NBLOCK = 8
BLKCH = NCHUNK // NBLOCK

_mesh = plsc.VectorSubcoreMesh(
    core_axis_name="c", subcore_axis_name="s", num_cores=NC, num_subcores=NS)


def _zero_vmem_rows(buf, nrows, ncols):
    z = jnp.zeros((16,), jnp.float32)

    def body(i, _):
        for j in range(ncols // 16):
            buf[i, pl.ds(j * 16, 16)] = z
        return 0

    lax.fori_loop(0, nrows, body, 0)


@functools.partial(
    pl.kernel,
    out_type=jax.ShapeDtypeStruct((NC * N_PAD,), jnp.float32),
    mesh=_mesh,
    scratch_types=[
        pltpu.VMEM((BLKCH, CHUNK), jnp.int32),
        pltpu.VMEM((CHUNK,), jnp.float32),
        pltpu.VMEM((ROWS_PER_TILE,), jnp.float32),
        pltpu.VMEM_SHARED((N_PAD,), jnp.float32),
    ],
)
def _deg_kernel(dst_hbm, out_hbm, didx_v, ones_v, zstage_v, acc):
    c = lax.axis_index("c")
    s = lax.axis_index("s")
    w = c * NS + s

    one = jnp.full((16,), 1.0, jnp.float32)
    zero = jnp.zeros((16,), jnp.float32)
    for j in range(CHUNK // 16):
        ones_v[pl.ds(j * 16, 16)] = one

    def zbody(i, _):
        zstage_v[pl.ds(i * 16, 16)] = zero
        return 0

    lax.fori_loop(0, ROWS_PER_TILE // 16, zbody, 0)
    pltpu.sync_copy(zstage_v, acc.at[pl.ds(s * ROWS_PER_TILE, ROWS_PER_TILE)])
    plsc.subcore_barrier()

    def body(k, _):
        pltpu.sync_copy(ones_v, acc.at[didx_v.at[k]], add=True)
        return 0

    for b in range(NBLOCK):
        pltpu.sync_copy(dst_hbm.at[w, b], didx_v)
        lax.fori_loop(0, BLKCH, body, 0)
    plsc.subcore_barrier()
    row0 = s * ROWS_PER_TILE
    pltpu.sync_copy(acc.at[pl.ds(row0, ROWS_PER_TILE)],
                    out_hbm.at[pl.ds(c * N_PAD + row0, ROWS_PER_TILE)])


@functools.partial(
    pl.kernel,
    out_type=jax.ShapeDtypeStruct((NC * N_PAD, D), jnp.float32),
    mesh=_mesh,
    scratch_types=[
        pltpu.VMEM((BLKCH, CHUNK), jnp.int32),
        pltpu.VMEM((BLKCH, CHUNK), jnp.int32),
        [pltpu.VMEM((CHUNK, D), jnp.float32)] * 3,
        [pltpu.SemaphoreType.DMA] * 3,
        pltpu.VMEM_SHARED((N_PAD, D), jnp.float32),
    ],
)
def _msg_kernel(table_hbm, src_hbm, dst_hbm, out_hbm, sidx, didx, rows,
                sem_g, acc):
    c = lax.axis_index("c")
    s = lax.axis_index("s")
    w = c * NS + s

    _zero_vmem_rows(rows[0], CHUNK, D)
    row0 = s * ROWS_PER_TILE
    for j in range(ROWS_PER_TILE // CHUNK):
        pltpu.sync_copy(rows[0], acc.at[pl.ds(row0 + j * CHUNK, CHUNK)])
    plsc.subcore_barrier()

    def gfire(k, r):
        pltpu.async_copy(table_hbm.at[sidx.at[k]], rows[r], sem_g[r])

    def gwait(k, r):
        pltpu.make_async_copy(table_hbm.at[sidx.at[k]], rows[r],
                              sem_g[r]).wait()

    def scat(k, r):
        pltpu.sync_copy(rows[r], acc.at[didx.at[k]], add=True)

    def body(j, _):
        k0 = 3 * j
        for t in range(3):
            gwait(k0 + t, t)
            scat(k0 + t, t)
            gfire(k0 + t + 2, (t + 2) % 3)
        return 0

    for b in range(NBLOCK):
        pltpu.sync_copy(src_hbm.at[w, b], sidx)
        pltpu.sync_copy(dst_hbm.at[w, b], didx)
        gfire(0, 0)
        gfire(1, 1)
        lax.fori_loop(0, (BLKCH - 4) // 3, body, 0)
        gwait(BLKCH - 4, 0)
        scat(BLKCH - 4, 0)
        gfire(BLKCH - 2, 2)
        gwait(BLKCH - 3, 1)
        scat(BLKCH - 3, 1)
        gfire(BLKCH - 1, 0)
        gwait(BLKCH - 2, 2)
        scat(BLKCH - 2, 2)
        gwait(BLKCH - 1, 0)
        scat(BLKCH - 1, 0)

    plsc.subcore_barrier()
    pltpu.sync_copy(acc.at[pl.ds(row0, ROWS_PER_TILE)],
                    out_hbm.at[pl.ds(c * N_PAD + row0, ROWS_PER_TILE)])


_BLK = 1024
_GRID = N_PAD // _BLK


def _dis_block(deg_ref):
    deg = deg_ref[0, :] + deg_ref[1, :] + 1.0
    return lax.rsqrt(deg)[:, None]


def _tc0_body(deg_ref, x_ref, xs_ref):
    xs_ref[...] = _dis_block(deg_ref) * x_ref[...]


def _tc1_body(deg_ref, msg_ref, xs_ref, w1_ref, b1_ref, a_ref, w2_ref,
              h2s_ref):
    dis = _dis_block(deg_ref)
    a = a_ref[0, 0]
    t = dis * (msg_ref[0] + msg_ref[1] + xs_ref[...])
    c1 = jnp.dot(t, w1_ref[...], preferred_element_type=jnp.float32)
    c1 = c1 + b1_ref[...]
    o1 = jnp.where(c1 >= 0, c1, a * c1)
    h2 = jnp.dot(o1, w2_ref[...], preferred_element_type=jnp.float32)
    h2s_ref[...] = dis * h2


def _tc2_body(deg_ref, msg_ref, h2s_ref, b2_ref, a_ref, f1w_ref, f1b_ref,
              f2w_ref, f2b_ref, out_ref, proj_ref):
    dis = _dis_block(deg_ref)
    a = a_ref[0, 0]
    t = dis * (msg_ref[0] + msg_ref[1] + h2s_ref[...]) + b2_ref[...]
    out = jnp.where(t >= 0, t, a * t)
    out_ref[...] = out
    p = jnp.dot(out, f1w_ref[...], preferred_element_type=jnp.float32)
    p = jnp.maximum(p + f1b_ref[...], 0.0)
    proj_ref[...] = jnp.dot(p, f2w_ref[...],
                            preferred_element_type=jnp.float32) + f2b_ref[...]


def _deg_spec():
    return pl.BlockSpec((2, _BLK), lambda i: (0, i))


def _row_spec(d=D):
    return pl.BlockSpec((_BLK, d), lambda i: (i, 0))


def _msg_spec():
    return pl.BlockSpec((2, _BLK, D), lambda i: (0, i, 0))


def _full_spec(shape):
    return pl.BlockSpec(shape, lambda i: tuple(0 for _ in shape))


def _smem_spec():
    return pl.BlockSpec(memory_space=pltpu.SMEM)


def kernel(x, edge_index, W1, b1, W2, b2, a, fc1_W, fc1_b, fc2_W, fc2_b):
    f32 = jnp.float32
    src = edge_index[0].astype(jnp.int32)
    dst = edge_index[1].astype(jnp.int32)
    pad = N + jnp.arange(E_PAD - src.shape[0], dtype=jnp.int32) % (N_PAD - N)
    src_p = jnp.concatenate([src, pad]).reshape(NW, NBLOCK, BLKCH, CHUNK)
    dst_p = jnp.concatenate([dst, pad]).reshape(NW, NBLOCK, BLKCH, CHUNK)

    deg_p = _deg_kernel(dst_p).reshape(NC, N_PAD)

    xs = pl.pallas_call(
        _tc0_body,
        grid=(_GRID,),
        in_specs=[_deg_spec(), _row_spec()],
        out_specs=_row_spec(),
        out_shape=jax.ShapeDtypeStruct((N_PAD, D), f32),
    )(deg_p, x)

    msg1 = _msg_kernel(xs, src_p, dst_p).reshape(NC, N_PAD, D)

    h2s = pl.pallas_call(
        _tc1_body,
        grid=(_GRID,),
        in_specs=[
            _deg_spec(), _msg_spec(), _row_spec(),
            _full_spec((D, D_HID)), _full_spec((1, D_HID)), _smem_spec(),
            _full_spec((D_HID, D)),
        ],
        out_specs=_row_spec(),
        out_shape=jax.ShapeDtypeStruct((N_PAD, D), f32),
    )(deg_p, msg1, xs, W1, b1.reshape(1, D_HID), a.reshape(1, 1), W2)

    msg2 = _msg_kernel(h2s, src_p, dst_p).reshape(NC, N_PAD, D)

    out, proj = pl.pallas_call(
        _tc2_body,
        grid=(_GRID,),
        in_specs=[
            _deg_spec(), _msg_spec(), _row_spec(),
            _full_spec((1, D)), _smem_spec(),
            _full_spec((D, D)), _full_spec((1, D)),
            _full_spec((D, D)), _full_spec((1, D)),
        ],
        out_specs=[_row_spec(), _row_spec()],
        out_shape=[
            jax.ShapeDtypeStruct((N, D), f32),
            jax.ShapeDtypeStruct((N, D), f32),
        ],
    )(deg_p, msg2, h2s, b2.reshape(1, D), a.reshape(1, 1), fc1_W,
      fc1_b.reshape(1, D), fc2_W, fc2_b.reshape(1, D))

    return (out, proj)


# speedup vs baseline: 1.2099x; 1.2099x over previous
import functools

import jax
import jax.numpy as jnp
from jax import lax
from jax.experimental import pallas as pl
from jax.experimental.pallas import tpu as pltpu
from jax.experimental.pallas import tpu_sc as plsc

N = 10000
D = 128
D_HID = 256

NC = 2
NS = 16
NW = NC * NS

N_PAD = 10240
ROWS_PER_TILE = N_PAD // NS
E_PAD = 327680
EPW = E_PAD // NW
CHUNK = 128
NCHUNK = EPW // CHUNK
NBLOCK = 2
BLKCH = NCHUNK // NBLOCK

_mesh = plsc.VectorSubcoreMesh(
    core_axis_name="c", subcore_axis_name="s", num_cores=NC, num_subcores=NS)


def _zero_vmem_rows(buf, nrows, ncols):
    z = jnp.zeros((16,), jnp.float32)

    def body(i, _):
        for j in range(ncols // 16):
            buf[i, pl.ds(j * 16, 16)] = z
        return 0

    lax.fori_loop(0, nrows, body, 0)


@functools.partial(
    pl.kernel,
    out_type=jax.ShapeDtypeStruct((NC * N_PAD,), jnp.float32),
    mesh=_mesh,
    scratch_types=[
        pltpu.VMEM((BLKCH, CHUNK), jnp.int32),
        pltpu.VMEM((CHUNK,), jnp.float32),
        pltpu.VMEM((ROWS_PER_TILE,), jnp.float32),
        pltpu.SemaphoreType.DMA,
        pltpu.VMEM_SHARED((N_PAD,), jnp.float32),
    ],
)
def _deg_kernel(dst_hbm, out_hbm, didx_v, ones_v, zstage_v, sem, acc):
    c = lax.axis_index("c")
    s = lax.axis_index("s")
    w = c * NS + s

    one = jnp.full((16,), 1.0, jnp.float32)
    zero = jnp.zeros((16,), jnp.float32)
    for j in range(CHUNK // 16):
        ones_v[pl.ds(j * 16, 16)] = one

    def zbody(i, _):
        zstage_v[pl.ds(i * 16, 16)] = zero
        return 0

    lax.fori_loop(0, ROWS_PER_TILE // 16, zbody, 0)
    pltpu.sync_copy(zstage_v, acc.at[pl.ds(s * ROWS_PER_TILE, ROWS_PER_TILE)])
    plsc.subcore_barrier()

    def body(k, _):
        pltpu.async_copy(ones_v, acc.at[didx_v.at[k]], sem, add=True)
        return 0

    def drain(k, _):
        pltpu.make_async_copy(ones_v, acc.at[didx_v.at[0]], sem).wait()
        return 0

    for b in range(NBLOCK):
        pltpu.sync_copy(dst_hbm.at[w, b], didx_v)
        lax.fori_loop(0, BLKCH, body, 0)
        lax.fori_loop(0, BLKCH, drain, 0)
    plsc.subcore_barrier()
    row0 = s * ROWS_PER_TILE
    pltpu.sync_copy(acc.at[pl.ds(row0, ROWS_PER_TILE)],
                    out_hbm.at[pl.ds(c * N_PAD + row0, ROWS_PER_TILE)])


@functools.partial(
    pl.kernel,
    out_type=jax.ShapeDtypeStruct((NC * N_PAD, D), jnp.float32),
    mesh=_mesh,
    scratch_types=[
        pltpu.VMEM((BLKCH, CHUNK), jnp.int32),
        pltpu.VMEM((BLKCH, CHUNK), jnp.int32),
        [pltpu.VMEM((CHUNK, D), jnp.float32)] * 2,
        [pltpu.SemaphoreType.DMA] * 2,
        pltpu.VMEM_SHARED((N_PAD, D), jnp.float32),
    ],
)
def _msg_kernel(table_hbm, src_hbm, dst_hbm, out_hbm, sidx, didx, rows,
                sem_g, acc):
    c = lax.axis_index("c")
    s = lax.axis_index("s")
    w = c * NS + s

    _zero_vmem_rows(rows[0], CHUNK, D)
    row0 = s * ROWS_PER_TILE
    for j in range(ROWS_PER_TILE // CHUNK):
        pltpu.sync_copy(rows[0], acc.at[pl.ds(row0 + j * CHUNK, CHUNK)])
    plsc.subcore_barrier()

    rows_a, rows_b = rows
    sem_a, sem_b = sem_g

    def body(j, _):
        k0 = 2 * j
        pltpu.async_copy(table_hbm.at[sidx.at[k0 + 1]], rows_b, sem_b)
        pltpu.make_async_copy(table_hbm.at[sidx.at[k0]], rows_a, sem_a).wait()
        pltpu.sync_copy(rows_a, acc.at[didx.at[k0]], add=True)
        k2 = jnp.minimum(k0 + 2, BLKCH - 1)
        pltpu.async_copy(table_hbm.at[sidx.at[k2]], rows_a, sem_a)
        pltpu.make_async_copy(table_hbm.at[sidx.at[k0 + 1]], rows_b,
                              sem_b).wait()
        pltpu.sync_copy(rows_b, acc.at[didx.at[k0 + 1]], add=True)
        return 0

    for b in range(NBLOCK):
        pltpu.sync_copy(src_hbm.at[w, b], sidx)
        pltpu.sync_copy(dst_hbm.at[w, b], didx)
        pltpu.async_copy(table_hbm.at[sidx.at[0]], rows_a, sem_a)
        lax.fori_loop(0, BLKCH // 2, body, 0)
        pltpu.make_async_copy(table_hbm.at[sidx.at[BLKCH - 1]], rows_a,
                              sem_a).wait()

    plsc.subcore_barrier()
    pltpu.sync_copy(acc.at[pl.ds(row0, ROWS_PER_TILE)],
                    out_hbm.at[pl.ds(c * N_PAD + row0, ROWS_PER_TILE)])


_BLK = 1024
_GRID = N_PAD // _BLK


def _dis_block(deg_ref):
    deg = deg_ref[0, :] + deg_ref[1, :] + 1.0
    return lax.rsqrt(deg)[:, None]


def _tc0_body(deg_ref, x_ref, xs_ref):
    xs_ref[...] = _dis_block(deg_ref) * x_ref[...]


def _tc1_body(deg_ref, msg_ref, xs_ref, w1_ref, b1_ref, a_ref, w2_ref,
              h2s_ref):
    dis = _dis_block(deg_ref)
    a = a_ref[0, 0]
    t = dis * (msg_ref[0] + msg_ref[1] + xs_ref[...])
    c1 = jnp.dot(t, w1_ref[...], preferred_element_type=jnp.float32)
    c1 = c1 + b1_ref[...]
    o1 = jnp.where(c1 >= 0, c1, a * c1)
    h2 = jnp.dot(o1, w2_ref[...], preferred_element_type=jnp.float32)
    h2s_ref[...] = dis * h2


def _tc2_body(deg_ref, msg_ref, h2s_ref, b2_ref, a_ref, f1w_ref, f1b_ref,
              f2w_ref, f2b_ref, out_ref, proj_ref):
    dis = _dis_block(deg_ref)
    a = a_ref[0, 0]
    t = dis * (msg_ref[0] + msg_ref[1] + h2s_ref[...]) + b2_ref[...]
    out = jnp.where(t >= 0, t, a * t)
    out_ref[...] = out
    p = jnp.dot(out, f1w_ref[...], preferred_element_type=jnp.float32)
    p = jnp.maximum(p + f1b_ref[...], 0.0)
    proj_ref[...] = jnp.dot(p, f2w_ref[...],
                            preferred_element_type=jnp.float32) + f2b_ref[...]


def _deg_spec():
    return pl.BlockSpec((2, _BLK), lambda i: (0, i))


def _row_spec(d=D):
    return pl.BlockSpec((_BLK, d), lambda i: (i, 0))


def _msg_spec():
    return pl.BlockSpec((2, _BLK, D), lambda i: (0, i, 0))


def _full_spec(shape):
    return pl.BlockSpec(shape, lambda i: tuple(0 for _ in shape))


def _smem_spec():
    return pl.BlockSpec(memory_space=pltpu.SMEM)


def kernel(x, edge_index, W1, b1, W2, b2, a, fc1_W, fc1_b, fc2_W, fc2_b):
    f32 = jnp.float32
    src = edge_index[0].astype(jnp.int32)
    dst = edge_index[1].astype(jnp.int32)
    pad = N + jnp.arange(E_PAD - src.shape[0], dtype=jnp.int32) % (N_PAD - N)
    src_p = jnp.concatenate([src, pad]).reshape(NW, NBLOCK, BLKCH, CHUNK)
    dst_p = jnp.concatenate([dst, pad]).reshape(NW, NBLOCK, BLKCH, CHUNK)

    deg_p = _deg_kernel(dst_p).reshape(NC, N_PAD)

    xs = pl.pallas_call(
        _tc0_body,
        grid=(_GRID,),
        in_specs=[_deg_spec(), _row_spec()],
        out_specs=_row_spec(),
        out_shape=jax.ShapeDtypeStruct((N_PAD, D), f32),
    )(deg_p, x)

    msg1 = _msg_kernel(xs, src_p, dst_p).reshape(NC, N_PAD, D)

    h2s = pl.pallas_call(
        _tc1_body,
        grid=(_GRID,),
        in_specs=[
            _deg_spec(), _msg_spec(), _row_spec(),
            _full_spec((D, D_HID)), _full_spec((1, D_HID)), _smem_spec(),
            _full_spec((D_HID, D)),
        ],
        out_specs=_row_spec(),
        out_shape=jax.ShapeDtypeStruct((N_PAD, D), f32),
    )(deg_p, msg1, xs, W1, b1.reshape(1, D_HID), a.reshape(1, 1), W2)

    msg2 = _msg_kernel(h2s, src_p, dst_p).reshape(NC, N_PAD, D)

    out, proj = pl.pallas_call(
        _tc2_body,
        grid=(_GRID,),
        in_specs=[
            _deg_spec(), _msg_spec(), _row_spec(),
            _full_spec((1, D)), _smem_spec(),
            _full_spec((D, D)), _full_spec((1, D)),
            _full_spec((D, D)), _full_spec((1, D)),
        ],
        out_specs=[_row_spec(), _row_spec()],
        out_shape=[
            jax.ShapeDtypeStruct((N, D), f32),
            jax.ShapeDtypeStruct((N, D), f32),
        ],
    )(deg_p, msg2, h2s, b2.reshape(1, D), a.reshape(1, 1), fc1_W,
      fc1_b.reshape(1, D), fc2_W, fc2_b.reshape(1, D))

    return (out, proj)
